# baseline (speedup 1.0000x reference)
import jax
import jax.numpy as jnp
from jax import lax
from jax.experimental import pallas as pl
from jax.experimental.pallas import tpu as pltpu

_R = 512
_NBUF = 12
_PREFETCH = 6


def kernel(x, pos_embed):
    B, S, D = x.shape
    x2 = x.reshape(B * S, D)
    nstep = (B * S) // _R
    s_blocks = S // _R

    def body(x_ref, pe_ref, o_ref, pe_v, xb, ld_sems, st_sems, pe_sem):
        pe_cp = pltpu.make_async_copy(pe_ref, pe_v, pe_sem)
        pe_cp.start()

        def start_load(j):
            cp = pltpu.make_async_copy(
                x_ref.at[pl.ds(j * _R, _R)], xb.at[j % _NBUF], ld_sems.at[j])
            cp.start()
            return cp

        def start_store(j):
            cp = pltpu.make_async_copy(
                xb.at[j % _NBUF], o_ref.at[pl.ds(j * _R, _R)], st_sems.at[j])
            cp.start()
            return cp

        loads = {}
        stores = {}
        for j in range(_PREFETCH):
            loads[j] = start_load(j)
        pe_cp.wait()

        for i in range(nstep):
            j = i + _PREFETCH
            if j < nstep:
                if j - _NBUF >= 0:
                    stores.pop(j - _NBUF).wait()
                loads[j] = start_load(j)

            loads.pop(i).wait()
            b = i % _NBUF
            prow = (i % s_blocks) * _R

            def row_body(k, carry, b=b, prow=prow):
                sl = pl.ds(k * 128, 128)
                psl = pl.ds(prow + k * 128, 128)
                xb[b, sl, :] = xb[b, sl, :] + pe_v[psl, :]
                return carry

            lax.fori_loop(0, _R // 128, row_body, 0)
            stores[i] = start_store(i)

        for i in sorted(stores):
            stores[i].wait()

    out = pl.pallas_call(
        body,
        in_specs=[
            pl.BlockSpec(memory_space=pl.ANY),
            pl.BlockSpec(memory_space=pl.ANY),
        ],
        out_specs=pl.BlockSpec(memory_space=pl.ANY),
        out_shape=jax.ShapeDtypeStruct((B * S, D), x.dtype),
        scratch_shapes=[
            pltpu.VMEM((S, D), jnp.float32),
            pltpu.VMEM((_NBUF, _R, D), jnp.float32),
            pltpu.SemaphoreType.DMA(((B * S) // _R,)),
            pltpu.SemaphoreType.DMA(((B * S) // _R,)),
            pltpu.SemaphoreType.DMA,
        ],
    )(x2, pos_embed)
    return out.reshape(B, S, D)


# speedup vs baseline: 1.0131x; 1.0131x over previous
import jax
import jax.numpy as jnp
from jax import lax
from jax.experimental import pallas as pl
from jax.experimental.pallas import tpu as pltpu

_R = 2048
_NBUF = 4
_PREFETCH = 2


def kernel(x, pos_embed):
    B, S, D = x.shape
    x2 = x.reshape(B * S, D)
    nstep = (B * S) // _R
    s_blocks = S // _R

    def body(x_ref, pe_ref, o_ref, pe_v, xb, ld_sems, st_sems, pe_sem):
        pe_cp = pltpu.make_async_copy(pe_ref, pe_v, pe_sem)
        pe_cp.start()

        def start_load(j):
            cp = pltpu.make_async_copy(
                x_ref.at[pl.ds(j * _R, _R)], xb.at[j % _NBUF], ld_sems.at[j])
            cp.start()
            return cp

        def start_store(j):
            cp = pltpu.make_async_copy(
                xb.at[j % _NBUF], o_ref.at[pl.ds(j * _R, _R)], st_sems.at[j])
            cp.start()
            return cp

        loads = {}
        stores = {}
        for j in range(_PREFETCH):
            loads[j] = start_load(j)
        pe_cp.wait()

        for i in range(nstep):
            j = i + _PREFETCH
            if j < nstep:
                if j - _NBUF >= 0:
                    stores.pop(j - _NBUF).wait()
                loads[j] = start_load(j)

            loads.pop(i).wait()
            b = i % _NBUF
            prow = (i % s_blocks) * _R

            def row_body(k, carry, b=b, prow=prow):
                sl = pl.ds(k * 128, 128)
                psl = pl.ds(prow + k * 128, 128)
                xb[b, sl, :] = xb[b, sl, :] + pe_v[psl, :]
                return carry

            lax.fori_loop(0, _R // 128, row_body, 0)
            stores[i] = start_store(i)

        for i in sorted(stores):
            stores[i].wait()

    out = pl.pallas_call(
        body,
        in_specs=[
            pl.BlockSpec(memory_space=pl.ANY),
            pl.BlockSpec(memory_space=pl.ANY),
        ],
        out_specs=pl.BlockSpec(memory_space=pl.ANY),
        out_shape=jax.ShapeDtypeStruct((B * S, D), x.dtype),
        scratch_shapes=[
            pltpu.VMEM((S, D), jnp.float32),
            pltpu.VMEM((_NBUF, _R, D), jnp.float32),
            pltpu.SemaphoreType.DMA(((B * S) // _R,)),
            pltpu.SemaphoreType.DMA(((B * S) // _R,)),
            pltpu.SemaphoreType.DMA,
        ],
    )(x2, pos_embed)
    return out.reshape(B, S, D)


# speedup vs baseline: 1.0179x; 1.0048x over previous
import jax
import jax.numpy as jnp
from jax import lax
from jax.experimental import pallas as pl
from jax.experimental.pallas import tpu as pltpu

_R = 1024
_NBUF = 6
_PREFETCH = 3


def kernel(x, pos_embed):
    B, S, D = x.shape
    x2 = x.reshape(B * S, D)
    nstep = (B * S) // _R
    s_blocks = S // _R

    def body(x_ref, pe_ref, o_ref, pe_v, xb, ld_sems, st_sems, pe_sem):
        pe_cp = pltpu.make_async_copy(pe_ref, pe_v, pe_sem)
        pe_cp.start()

        def start_load(j):
            cp = pltpu.make_async_copy(
                x_ref.at[pl.ds(j * _R, _R)], xb.at[j % _NBUF], ld_sems.at[j])
            cp.start()
            return cp

        def start_store(j):
            cp = pltpu.make_async_copy(
                xb.at[j % _NBUF], o_ref.at[pl.ds(j * _R, _R)], st_sems.at[j])
            cp.start()
            return cp

        loads = {}
        stores = {}
        for j in range(_PREFETCH):
            loads[j] = start_load(j)
        pe_cp.wait()

        for i in range(nstep):
            j = i + _PREFETCH
            if j < nstep:
                if j - _NBUF >= 0:
                    stores.pop(j - _NBUF).wait()
                loads[j] = start_load(j)

            loads.pop(i).wait()
            b = i % _NBUF
            prow = (i % s_blocks) * _R

            def row_body(k, carry, b=b, prow=prow):
                sl = pl.ds(k * 128, 128)
                psl = pl.ds(prow + k * 128, 128)
                xb[b, sl, :] = xb[b, sl, :] + pe_v[psl, :]
                return carry

            lax.fori_loop(0, _R // 128, row_body, 0)
            stores[i] = start_store(i)

        for i in sorted(stores):
            stores[i].wait()

    out = pl.pallas_call(
        body,
        in_specs=[
            pl.BlockSpec(memory_space=pl.ANY),
            pl.BlockSpec(memory_space=pl.ANY),
        ],
        out_specs=pl.BlockSpec(memory_space=pl.ANY),
        out_shape=jax.ShapeDtypeStruct((B * S, D), x.dtype),
        scratch_shapes=[
            pltpu.VMEM((S, D), jnp.float32),
            pltpu.VMEM((_NBUF, _R, D), jnp.float32),
            pltpu.SemaphoreType.DMA(((B * S) // _R,)),
            pltpu.SemaphoreType.DMA(((B * S) // _R,)),
            pltpu.SemaphoreType.DMA,
        ],
    )(x2, pos_embed)
    return out.reshape(B, S, D)
